# baseline (speedup 1.0000x reference)
import functools

import jax
import jax.numpy as jnp
from jax import lax
from jax.experimental import pallas as pl
from jax.experimental.pallas import tpu as pltpu
from jax.experimental.pallas import tpu_sc as plsc

N_FTS = 100000
D = 256
B = 16384
L = 32

NC = 2
NS = 16
NW = NC * NS
LANES = 16

SAMPLES = 2 * B
SPW = SAMPLES // NW
CHUNK = 2
NBUF = 4
SS = 256
SC_CHUNKS = SS // CHUNK


def _ft_body(ics_hbm, vals_hbm, table_hbm, out_hbm, idxs_v, vals_v,
             rows_v, accs_v, sems, osems):
    wid = lax.axis_index("s") * NC + lax.axis_index("c")
    base = wid * SPW
    RL = CHUNK * L

    def start_gather(g, b):
        pltpu.async_copy(
            table_hbm.at[idxs_v.at[pl.ds(g * RL, RL)]],
            rows_v.at[pl.ds(b * RL, RL)],
            sems[b],
        )

    def wait_gather(b):
        pltpu.make_async_copy(
            table_hbm.at[pl.ds(0, RL)], rows_v.at[pl.ds(b * RL, RL)], sems[b]
        ).wait()

    def compute_chunk(g, b, s0):
        def sample_body(i, carry2):
            r0 = (g * CHUNK + i) * L
            v0 = vals_v[pl.ds(r0, LANES)]
            v1 = vals_v[pl.ds(r0 + LANES, LANES)]
            accs = [jnp.zeros((LANES,), jnp.float32) for _ in range(D // LANES)]
            for l in range(L):
                vv = v0 if l < LANES else v1
                val = lax.index_in_dim(vv, l % LANES, 0, keepdims=False)
                r = b * RL + i * L + l
                for j in range(D // LANES):
                    accs[j] = accs[j] + rows_v[r, pl.ds(j * LANES, LANES)] * val
            for j in range(D // LANES):
                accs_v[b * CHUNK + i, pl.ds(j * LANES, LANES)] = accs[j]
            return carry2

        lax.fori_loop(0, CHUNK, sample_body, 0)
        pltpu.async_copy(
            accs_v.at[pl.ds(b * CHUNK, CHUNK)],
            out_hbm.at[pl.ds(s0 + g * CHUNK, CHUNK)],
            osems[b],
        )

    def wait_out(b):
        pltpu.make_async_copy(
            accs_v.at[pl.ds(b * CHUNK, CHUNK)],
            out_hbm.at[pl.ds(0, CHUNK)],
            osems[b],
        ).wait()

    def super_body(sidx, carry):
        s0 = base + sidx * SS
        pltpu.sync_copy(ics_hbm.at[pl.ds(s0 * L, SS * L)], idxs_v)
        pltpu.sync_copy(vals_hbm.at[pl.ds(s0 * L, SS * L)], vals_v)
        for b in range(NBUF - 1):
            start_gather(b, b)

        def quad_body(q, carry2):
            for b in range(NBUF):
                g = NBUF * q + b
                wait_gather(b)
                nxt = g + NBUF - 1

                @pl.when(nxt < SC_CHUNKS)
                def _():
                    start_gather(nxt, (b + NBUF - 1) % NBUF)

                @pl.when(q > 0)
                def _():
                    wait_out(b)

                compute_chunk(g, b, s0)
            return carry2

        lax.fori_loop(0, SC_CHUNKS // NBUF, quad_body, 0)
        for b in range(NBUF):
            wait_out(b)
        return carry

    lax.fori_loop(0, SPW // SS, super_body, 0)


def _feature_transform(ics_flat, vals_flat, table):
    mesh = plsc.VectorSubcoreMesh(core_axis_name="c", subcore_axis_name="s")
    return pl.kernel(
        _ft_body,
        out_type=jax.ShapeDtypeStruct((SAMPLES, D), jnp.float32),
        mesh=mesh,
        scratch_types=[
            pltpu.VMEM((SS * L,), jnp.int32),
            pltpu.VMEM((SS * L,), jnp.float32),
            pltpu.VMEM((NBUF * CHUNK * L, D), jnp.float32),
            pltpu.VMEM((NBUF * CHUNK, D), jnp.float32),
            [pltpu.SemaphoreType.DMA for _ in range(NBUF)],
            [pltpu.SemaphoreType.DMA for _ in range(NBUF)],
        ],
        name="nnue_feature_transform",
    )(ics_flat, vals_flat, table)


def _mlp_body(wf_ref, bf_ref, s_ref, bft_ref, W1_ref, b1_ref, W2_ref, b2_ref,
              Wo_ref, bo_ref, o_ref):
    bft = bft_ref[...]
    wf = wf_ref[...] + bft
    bf = bf_ref[...] + bft
    s = s_ref[...]
    x1 = jnp.clip((1.0 - s) * wf + s * bf, 0.0, 1.0)
    x2 = jnp.clip((1.0 - s) * bf + s * wf, 0.0, 1.0)
    dn = (((1,), (1,)), ((), ()))
    h = lax.dot_general(x1, W1_ref[:, :D], dn, preferred_element_type=jnp.float32)
    h += lax.dot_general(x2, W1_ref[:, D:], dn, preferred_element_type=jnp.float32)
    h = jnp.clip(h + b1_ref[...], 0.0, 1.0)
    h = lax.dot_general(h, W2_ref[...], dn, preferred_element_type=jnp.float32)
    h = jnp.clip(h + b2_ref[...], 0.0, 1.0)
    o_ref[...] = jnp.sum(h * Wo_ref[...], axis=1, keepdims=True) + bo_ref[...]


def _mlp_head(fts, stm, b_ft, W1, b1, W2, b2, Wo, bo):
    BB = 2048
    grid = (B // BB,)
    return pl.pallas_call(
        _mlp_body,
        grid=grid,
        in_specs=[
            pl.BlockSpec((BB, D), lambda i: (i, 0)),
            pl.BlockSpec((BB, D), lambda i: (B // BB + i, 0)),
            pl.BlockSpec((BB, 1), lambda i: (i, 0)),
            pl.BlockSpec((1, D), lambda i: (0, 0)),
            pl.BlockSpec((32, 2 * D), lambda i: (0, 0)),
            pl.BlockSpec((1, 32), lambda i: (0, 0)),
            pl.BlockSpec((32, 32), lambda i: (0, 0)),
            pl.BlockSpec((1, 32), lambda i: (0, 0)),
            pl.BlockSpec((1, 32), lambda i: (0, 0)),
            pl.BlockSpec((1, 1), lambda i: (0, 0)),
        ],
        out_specs=pl.BlockSpec((BB, 1), lambda i: (i, 0)),
        out_shape=jax.ShapeDtypeStruct((B, 1), jnp.float32),
    )(fts, fts, stm, b_ft, W1, b1, W2, b2, Wo, bo)


def kernel(wft_ics, wft_vals, bft_ics, bft_vals, stm, W_ft, b_ft, W1, b1, W2, b2, Wo, bo):
    ics_flat = jnp.concatenate([wft_ics, bft_ics]).reshape(-1)
    vals_flat = jnp.concatenate([wft_vals, bft_vals]).reshape(-1)
    fts = _feature_transform(ics_flat, vals_flat, W_ft)
    return _mlp_head(
        fts, stm,
        b_ft.reshape(1, D),
        W1, b1.reshape(1, 32), W2, b2.reshape(1, 32),
        Wo.reshape(1, 32), bo.reshape(1, 1),
    )


# speedup vs baseline: 1.6648x; 1.6648x over previous
import functools

import jax
import jax.numpy as jnp
from jax import lax
from jax.experimental import pallas as pl
from jax.experimental.pallas import tpu as pltpu
from jax.experimental.pallas import tpu_sc as plsc

N_FTS = 100000
D = 256
B = 16384
L = 32

NC = 2
NS = 16
NW = NC * NS
LANES = 16

SAMPLES = 2 * B
SPW = SAMPLES // NW
CHUNK = 4
NBUF = 3
SS = 128
SC_CHUNKS = SS // CHUNK


def _ft_body(ics_hbm, vals_hbm, table_hbm, out_hbm, idxs_v, vals_v,
             rows_v, accs_v, sems, osems):
    wid = lax.axis_index("s") * NC + lax.axis_index("c")
    base = wid * SPW
    RL = CHUNK * L

    def start_gather(g, b):
        pltpu.async_copy(
            table_hbm.at[idxs_v.at[pl.ds(g * RL, RL)]],
            rows_v.at[pl.ds(b * RL, RL)],
            sems[b],
        )

    def wait_gather(b):
        pltpu.make_async_copy(
            table_hbm.at[pl.ds(0, RL)], rows_v.at[pl.ds(b * RL, RL)], sems[b]
        ).wait()

    def compute_chunk(g, b, s0):
        def sample_body(i, carry2):
            r0 = (g * CHUNK + i) * L
            v0 = vals_v[pl.ds(r0, LANES)]
            v1 = vals_v[pl.ds(r0 + LANES, LANES)]
            rbase = b * RL + i * L

            def j_body(j, carry3):
                col = pl.multiple_of(j * LANES, LANES)
                acc = jnp.zeros((LANES,), jnp.float32)
                for l in range(L):
                    vv = v0 if l < LANES else v1
                    val = lax.index_in_dim(vv, l % LANES, 0, keepdims=False)
                    acc = acc + rows_v[rbase + l, pl.ds(col, LANES)] * val
                accs_v[b * CHUNK + i, pl.ds(col, LANES)] = acc
                return carry3

            lax.fori_loop(0, D // LANES, j_body, 0)
            return carry2

        lax.fori_loop(0, CHUNK, sample_body, 0)
        pltpu.async_copy(
            accs_v.at[pl.ds(b * CHUNK, CHUNK)],
            out_hbm.at[pl.ds(s0 + g * CHUNK, CHUNK)],
            osems[b],
        )

    def wait_out(b):
        pltpu.make_async_copy(
            accs_v.at[pl.ds(b * CHUNK, CHUNK)],
            out_hbm.at[pl.ds(0, CHUNK)],
            osems[b],
        ).wait()

    def super_body(sidx, carry):
        s0 = base + sidx * SS
        pltpu.sync_copy(ics_hbm.at[pl.ds(s0 * L, SS * L)], idxs_v)
        pltpu.sync_copy(vals_hbm.at[pl.ds(s0 * L, SS * L)], vals_v)
        for b in range(NBUF - 1):
            start_gather(b, b)

        def step(g, b, first_round):
            wait_gather(b)
            nxt = g + NBUF - 1

            @pl.when(nxt < SC_CHUNKS)
            def _():
                start_gather(nxt, (b + NBUF - 1) % NBUF)

            @pl.when(jnp.logical_not(first_round))
            def _():
                wait_out(b)

            compute_chunk(g, b, s0)

        def ring_body(q, carry2):
            for b in range(NBUF):
                step(NBUF * q + b, b, q < 1)
            return carry2

        n_full = SC_CHUNKS // NBUF
        lax.fori_loop(0, n_full, ring_body, 0)
        for b in range(SC_CHUNKS - n_full * NBUF):
            step(jnp.int32(n_full * NBUF + b), b, jnp.bool_(False))
        for b in range(NBUF):
            wait_out(b)
        return carry

    lax.fori_loop(0, SPW // SS, super_body, 0)


def _feature_transform(ics_flat, vals_flat, table):
    mesh = plsc.VectorSubcoreMesh(core_axis_name="c", subcore_axis_name="s")
    return pl.kernel(
        _ft_body,
        out_type=jax.ShapeDtypeStruct((SAMPLES, D), jnp.float32),
        mesh=mesh,
        scratch_types=[
            pltpu.VMEM((SS * L,), jnp.int32),
            pltpu.VMEM((SS * L,), jnp.float32),
            pltpu.VMEM((NBUF * CHUNK * L, D), jnp.float32),
            pltpu.VMEM((NBUF * CHUNK, D), jnp.float32),
            [pltpu.SemaphoreType.DMA for _ in range(NBUF)],
            [pltpu.SemaphoreType.DMA for _ in range(NBUF)],
        ],
        name="nnue_feature_transform",
    )(ics_flat, vals_flat, table)


def _mlp_body(wf_ref, bf_ref, s_ref, bft_ref, W1_ref, b1_ref, W2_ref, b2_ref,
              Wo_ref, bo_ref, o_ref):
    bft = bft_ref[...]
    wf = wf_ref[...] + bft
    bf = bf_ref[...] + bft
    s = s_ref[...]
    x1 = jnp.clip((1.0 - s) * wf + s * bf, 0.0, 1.0)
    x2 = jnp.clip((1.0 - s) * bf + s * wf, 0.0, 1.0)
    dn = (((1,), (1,)), ((), ()))
    h = lax.dot_general(x1, W1_ref[:, :D], dn, preferred_element_type=jnp.float32)
    h += lax.dot_general(x2, W1_ref[:, D:], dn, preferred_element_type=jnp.float32)
    h = jnp.clip(h + b1_ref[...], 0.0, 1.0)
    h = lax.dot_general(h, W2_ref[...], dn, preferred_element_type=jnp.float32)
    h = jnp.clip(h + b2_ref[...], 0.0, 1.0)
    o_ref[...] = jnp.sum(h * Wo_ref[...], axis=1, keepdims=True) + bo_ref[...]


def _mlp_head(fts, stm, b_ft, W1, b1, W2, b2, Wo, bo):
    BB = 2048
    grid = (B // BB,)
    return pl.pallas_call(
        _mlp_body,
        grid=grid,
        in_specs=[
            pl.BlockSpec((BB, D), lambda i: (i, 0)),
            pl.BlockSpec((BB, D), lambda i: (B // BB + i, 0)),
            pl.BlockSpec((BB, 1), lambda i: (i, 0)),
            pl.BlockSpec((1, D), lambda i: (0, 0)),
            pl.BlockSpec((32, 2 * D), lambda i: (0, 0)),
            pl.BlockSpec((1, 32), lambda i: (0, 0)),
            pl.BlockSpec((32, 32), lambda i: (0, 0)),
            pl.BlockSpec((1, 32), lambda i: (0, 0)),
            pl.BlockSpec((1, 32), lambda i: (0, 0)),
            pl.BlockSpec((1, 1), lambda i: (0, 0)),
        ],
        out_specs=pl.BlockSpec((BB, 1), lambda i: (i, 0)),
        out_shape=jax.ShapeDtypeStruct((B, 1), jnp.float32),
    )(fts, fts, stm, b_ft, W1, b1, W2, b2, Wo, bo)


def kernel(wft_ics, wft_vals, bft_ics, bft_vals, stm, W_ft, b_ft, W1, b1, W2, b2, Wo, bo):
    ics_flat = jnp.concatenate([wft_ics, bft_ics]).reshape(-1)
    vals_flat = jnp.concatenate([wft_vals, bft_vals]).reshape(-1)
    fts = _feature_transform(ics_flat, vals_flat, W_ft)
    return _mlp_head(
        fts, stm,
        b_ft.reshape(1, D),
        W1, b1.reshape(1, 32), W2, b2.reshape(1, 32),
        Wo.reshape(1, 32), bo.reshape(1, 1),
    )


# speedup vs baseline: 2.3771x; 1.4279x over previous
import functools

import jax
import jax.numpy as jnp
from jax import lax
from jax.experimental import pallas as pl
from jax.experimental.pallas import tpu as pltpu
from jax.experimental.pallas import tpu_sc as plsc

N_FTS = 100000
D = 256
B = 16384
L = 32

NC = 2
NS = 16
NW = NC * NS
LANES = 16

SAMPLES = 2 * B
SPW = SAMPLES // NW
CHUNK = 4
NBUF = 3
SS = 128
SC_CHUNKS = SS // CHUNK


def _ft_body(ics_hbm, vals_hbm, table_hbm, out_hbm, idxs_v, vals_v,
             rows_v, accs_v, sems, osems):
    wid = lax.axis_index("s") * NC + lax.axis_index("c")
    base = wid * SPW
    RL = CHUNK * L

    def start_gather(g, b):
        pltpu.async_copy(
            table_hbm.at[idxs_v.at[pl.ds(g * RL, RL)]],
            rows_v.at[pl.ds(b * RL, RL)],
            sems[b],
        )

    def wait_gather(b):
        pltpu.make_async_copy(
            table_hbm.at[pl.ds(0, RL)], rows_v.at[pl.ds(b * RL, RL)], sems[b]
        ).wait()

    def compute_chunk(g, b, s0):
        def sample_body(i, carry2):
            r0 = (g * CHUNK + i) * L
            v0 = vals_v[pl.ds(r0, LANES)]
            v1 = vals_v[pl.ds(r0 + LANES, LANES)]
            rbase = b * RL + i * L

            def j_body(j, carry3):
                col = pl.multiple_of(j * LANES, LANES)
                part = [jnp.zeros((LANES,), jnp.float32) for _ in range(4)]
                for l in range(L):
                    vv = v0 if l < LANES else v1
                    val = lax.index_in_dim(vv, l % LANES, 0, keepdims=False)
                    part[l % 4] = part[l % 4] + rows_v[rbase + l, pl.ds(col, LANES)] * val
                acc = (part[0] + part[1]) + (part[2] + part[3])
                accs_v[b * CHUNK + i, pl.ds(col, LANES)] = acc
                return carry3

            lax.fori_loop(0, D // LANES, j_body, 0)
            return carry2

        lax.fori_loop(0, CHUNK, sample_body, 0)
        pltpu.async_copy(
            accs_v.at[pl.ds(b * CHUNK, CHUNK)],
            out_hbm.at[pl.ds(s0 + g * CHUNK, CHUNK)],
            osems[b],
        )

    def wait_out(b):
        pltpu.make_async_copy(
            accs_v.at[pl.ds(b * CHUNK, CHUNK)],
            out_hbm.at[pl.ds(0, CHUNK)],
            osems[b],
        ).wait()

    def super_body(sidx, carry):
        s0 = base + sidx * SS
        pltpu.sync_copy(ics_hbm.at[pl.ds(s0 * L, SS * L)], idxs_v)
        pltpu.sync_copy(vals_hbm.at[pl.ds(s0 * L, SS * L)], vals_v)
        for b in range(NBUF - 1):
            start_gather(b, b)

        def step(g, b, first_round):
            wait_gather(b)
            nxt = g + NBUF - 1

            @pl.when(nxt < SC_CHUNKS)
            def _():
                start_gather(nxt, (b + NBUF - 1) % NBUF)

            @pl.when(jnp.logical_not(first_round))
            def _():
                wait_out(b)

            compute_chunk(g, b, s0)

        def ring_body(q, carry2):
            for b in range(NBUF):
                step(NBUF * q + b, b, q < 1)
            return carry2

        n_full = SC_CHUNKS // NBUF
        lax.fori_loop(0, n_full, ring_body, 0)
        for b in range(SC_CHUNKS - n_full * NBUF):
            step(jnp.int32(n_full * NBUF + b), b, jnp.bool_(False))
        for b in range(NBUF):
            wait_out(b)
        return carry

    lax.fori_loop(0, SPW // SS, super_body, 0)


def _feature_transform(ics_flat, vals_flat, table):
    mesh = plsc.VectorSubcoreMesh(core_axis_name="c", subcore_axis_name="s")
    return pl.kernel(
        _ft_body,
        out_type=jax.ShapeDtypeStruct((SAMPLES, D), jnp.float32),
        mesh=mesh,
        scratch_types=[
            pltpu.VMEM((SS * L,), jnp.int32),
            pltpu.VMEM((SS * L,), jnp.float32),
            pltpu.VMEM((NBUF * CHUNK * L, D), jnp.float32),
            pltpu.VMEM((NBUF * CHUNK, D), jnp.float32),
            [pltpu.SemaphoreType.DMA for _ in range(NBUF)],
            [pltpu.SemaphoreType.DMA for _ in range(NBUF)],
        ],
        name="nnue_feature_transform",
    )(ics_flat, vals_flat, table)


def _mlp_body(wf_ref, bf_ref, s_ref, bft_ref, W1_ref, b1_ref, W2_ref, b2_ref,
              Wo_ref, bo_ref, o_ref):
    bft = bft_ref[...]
    wf = wf_ref[...] + bft
    bf = bf_ref[...] + bft
    s = s_ref[...]
    x1 = jnp.clip((1.0 - s) * wf + s * bf, 0.0, 1.0)
    x2 = jnp.clip((1.0 - s) * bf + s * wf, 0.0, 1.0)
    dn = (((1,), (1,)), ((), ()))
    h = lax.dot_general(x1, W1_ref[:, :D], dn, preferred_element_type=jnp.float32)
    h += lax.dot_general(x2, W1_ref[:, D:], dn, preferred_element_type=jnp.float32)
    h = jnp.clip(h + b1_ref[...], 0.0, 1.0)
    h = lax.dot_general(h, W2_ref[...], dn, preferred_element_type=jnp.float32)
    h = jnp.clip(h + b2_ref[...], 0.0, 1.0)
    o_ref[...] = jnp.sum(h * Wo_ref[...], axis=1, keepdims=True) + bo_ref[...]


def _mlp_head(fts, stm, b_ft, W1, b1, W2, b2, Wo, bo):
    BB = 2048
    grid = (B // BB,)
    return pl.pallas_call(
        _mlp_body,
        grid=grid,
        in_specs=[
            pl.BlockSpec((BB, D), lambda i: (i, 0)),
            pl.BlockSpec((BB, D), lambda i: (B // BB + i, 0)),
            pl.BlockSpec((BB, 1), lambda i: (i, 0)),
            pl.BlockSpec((1, D), lambda i: (0, 0)),
            pl.BlockSpec((32, 2 * D), lambda i: (0, 0)),
            pl.BlockSpec((1, 32), lambda i: (0, 0)),
            pl.BlockSpec((32, 32), lambda i: (0, 0)),
            pl.BlockSpec((1, 32), lambda i: (0, 0)),
            pl.BlockSpec((1, 32), lambda i: (0, 0)),
            pl.BlockSpec((1, 1), lambda i: (0, 0)),
        ],
        out_specs=pl.BlockSpec((BB, 1), lambda i: (i, 0)),
        out_shape=jax.ShapeDtypeStruct((B, 1), jnp.float32),
    )(fts, fts, stm, b_ft, W1, b1, W2, b2, Wo, bo)


def kernel(wft_ics, wft_vals, bft_ics, bft_vals, stm, W_ft, b_ft, W1, b1, W2, b2, Wo, bo):
    ics_flat = jnp.concatenate([wft_ics, bft_ics]).reshape(-1)
    vals_flat = jnp.concatenate([wft_vals, bft_vals]).reshape(-1)
    fts = _feature_transform(ics_flat, vals_flat, W_ft)
    return _mlp_head(
        fts, stm,
        b_ft.reshape(1, D),
        W1, b1.reshape(1, 32), W2, b2.reshape(1, 32),
        Wo.reshape(1, 32), bo.reshape(1, 1),
    )


# speedup vs baseline: 2.4545x; 1.0326x over previous
import functools

import jax
import jax.numpy as jnp
from jax import lax
from jax.experimental import pallas as pl
from jax.experimental.pallas import tpu as pltpu
from jax.experimental.pallas import tpu_sc as plsc

N_FTS = 100000
D = 256
B = 16384
L = 32

NC = 2
NS = 16
NW = NC * NS
LANES = 16

SAMPLES = 2 * B
SPW = SAMPLES // NW
CHUNK = 4
NBUF = 3
SS = 256
SC_CHUNKS = SS // CHUNK


def _ft_body(ics_hbm, vals_hbm, table_hbm, out_hbm, idxs_v, vals_v,
             rows_v, accs_v, sems, osems):
    wid = lax.axis_index("s") * NC + lax.axis_index("c")
    base = wid * SPW
    RL = CHUNK * L

    def start_gather(g, b):
        pltpu.async_copy(
            table_hbm.at[idxs_v.at[pl.ds(g * RL, RL)]],
            rows_v.at[pl.ds(b * RL, RL)],
            sems[b],
        )

    def wait_gather(b):
        pltpu.make_async_copy(
            table_hbm.at[pl.ds(0, RL)], rows_v.at[pl.ds(b * RL, RL)], sems[b]
        ).wait()

    def compute_chunk(g, b, s0):
        def sample_body(i, carry2):
            r0 = (g * CHUNK + i) * L
            v0 = vals_v[pl.ds(r0, LANES)]
            v1 = vals_v[pl.ds(r0 + LANES, LANES)]
            rbase = b * RL + i * L

            def j_body(j, carry3):
                col = pl.multiple_of(j * LANES, LANES)
                part = [jnp.zeros((LANES,), jnp.float32) for _ in range(4)]
                for l in range(L):
                    vv = v0 if l < LANES else v1
                    val = lax.index_in_dim(vv, l % LANES, 0, keepdims=False)
                    part[l % 4] = part[l % 4] + rows_v[rbase + l, pl.ds(col, LANES)] * val
                acc = (part[0] + part[1]) + (part[2] + part[3])
                accs_v[b * CHUNK + i, pl.ds(col, LANES)] = acc
                return carry3

            lax.fori_loop(0, D // LANES, j_body, 0)
            return carry2

        lax.fori_loop(0, CHUNK, sample_body, 0)
        pltpu.async_copy(
            accs_v.at[pl.ds(b * CHUNK, CHUNK)],
            out_hbm.at[pl.ds(s0 + g * CHUNK, CHUNK)],
            osems[b],
        )

    def wait_out(b):
        pltpu.make_async_copy(
            accs_v.at[pl.ds(b * CHUNK, CHUNK)],
            out_hbm.at[pl.ds(0, CHUNK)],
            osems[b],
        ).wait()

    def super_body(sidx, carry):
        s0 = base + sidx * SS
        pltpu.sync_copy(ics_hbm.at[pl.ds(s0 * L, SS * L)], idxs_v)
        pltpu.sync_copy(vals_hbm.at[pl.ds(s0 * L, SS * L)], vals_v)
        for b in range(NBUF - 1):
            start_gather(b, b)

        def step(g, b, first_round):
            wait_gather(b)
            nxt = g + NBUF - 1

            @pl.when(nxt < SC_CHUNKS)
            def _():
                start_gather(nxt, (b + NBUF - 1) % NBUF)

            @pl.when(jnp.logical_not(first_round))
            def _():
                wait_out(b)

            compute_chunk(g, b, s0)

        def ring_body(q, carry2):
            for b in range(NBUF):
                step(NBUF * q + b, b, q < 1)
            return carry2

        n_full = SC_CHUNKS // NBUF
        lax.fori_loop(0, n_full, ring_body, 0)
        for b in range(SC_CHUNKS - n_full * NBUF):
            step(jnp.int32(n_full * NBUF + b), b, jnp.bool_(False))
        for b in range(NBUF):
            wait_out(b)
        return carry

    lax.fori_loop(0, SPW // SS, super_body, 0)


def _feature_transform(ics_flat, vals_flat, table):
    mesh = plsc.VectorSubcoreMesh(core_axis_name="c", subcore_axis_name="s")
    return pl.kernel(
        _ft_body,
        out_type=jax.ShapeDtypeStruct((SAMPLES, D), jnp.float32),
        mesh=mesh,
        scratch_types=[
            pltpu.VMEM((SS * L,), jnp.int32),
            pltpu.VMEM((SS * L,), jnp.float32),
            pltpu.VMEM((NBUF * CHUNK * L, D), jnp.float32),
            pltpu.VMEM((NBUF * CHUNK, D), jnp.float32),
            [pltpu.SemaphoreType.DMA for _ in range(NBUF)],
            [pltpu.SemaphoreType.DMA for _ in range(NBUF)],
        ],
        name="nnue_feature_transform",
    )(ics_flat, vals_flat, table)


def _mlp_body(wf_ref, bf_ref, s_ref, bft_ref, W1_ref, b1_ref, W2_ref, b2_ref,
              Wo_ref, bo_ref, o_ref):
    bft = bft_ref[...]
    wf = wf_ref[...] + bft
    bf = bf_ref[...] + bft
    s = s_ref[...]
    x1 = jnp.clip((1.0 - s) * wf + s * bf, 0.0, 1.0)
    x2 = jnp.clip((1.0 - s) * bf + s * wf, 0.0, 1.0)
    dn = (((1,), (1,)), ((), ()))
    h = lax.dot_general(x1, W1_ref[:, :D], dn, preferred_element_type=jnp.float32)
    h += lax.dot_general(x2, W1_ref[:, D:], dn, preferred_element_type=jnp.float32)
    h = jnp.clip(h + b1_ref[...], 0.0, 1.0)
    h = lax.dot_general(h, W2_ref[...], dn, preferred_element_type=jnp.float32)
    h = jnp.clip(h + b2_ref[...], 0.0, 1.0)
    o_ref[...] = jnp.sum(h * Wo_ref[...], axis=1, keepdims=True) + bo_ref[...]


def _mlp_head(fts, stm, b_ft, W1, b1, W2, b2, Wo, bo):
    BB = 2048
    grid = (B // BB,)
    return pl.pallas_call(
        _mlp_body,
        grid=grid,
        in_specs=[
            pl.BlockSpec((BB, D), lambda i: (i, 0)),
            pl.BlockSpec((BB, D), lambda i: (B // BB + i, 0)),
            pl.BlockSpec((BB, 1), lambda i: (i, 0)),
            pl.BlockSpec((1, D), lambda i: (0, 0)),
            pl.BlockSpec((32, 2 * D), lambda i: (0, 0)),
            pl.BlockSpec((1, 32), lambda i: (0, 0)),
            pl.BlockSpec((32, 32), lambda i: (0, 0)),
            pl.BlockSpec((1, 32), lambda i: (0, 0)),
            pl.BlockSpec((1, 32), lambda i: (0, 0)),
            pl.BlockSpec((1, 1), lambda i: (0, 0)),
        ],
        out_specs=pl.BlockSpec((BB, 1), lambda i: (i, 0)),
        out_shape=jax.ShapeDtypeStruct((B, 1), jnp.float32),
    )(fts, fts, stm, b_ft, W1, b1, W2, b2, Wo, bo)


def kernel(wft_ics, wft_vals, bft_ics, bft_vals, stm, W_ft, b_ft, W1, b1, W2, b2, Wo, bo):
    ics_flat = jnp.concatenate([wft_ics, bft_ics]).reshape(-1)
    vals_flat = jnp.concatenate([wft_vals, bft_vals]).reshape(-1)
    fts = _feature_transform(ics_flat, vals_flat, W_ft)
    return _mlp_head(
        fts, stm,
        b_ft.reshape(1, D),
        W1, b1.reshape(1, 32), W2, b2.reshape(1, 32),
        Wo.reshape(1, 32), bo.reshape(1, 1),
    )
